# Initial kernel scaffold; baseline (speedup 1.0000x reference)
import jax
import jax.numpy as jnp
from jax.experimental import pallas as pl


def kernel(x):
    raise NotImplementedError("write your pallas kernel here")


# speedup vs baseline: 2.4863x; 2.4863x over previous
import functools

import jax
import jax.numpy as jnp
from jax import lax
from jax.experimental import pallas as pl
from jax.experimental.pallas import tpu as pltpu
from jax.experimental.pallas import tpu_sc as plsc

L = 20
B = 1024
T = 1000
N = L * B
NC = 2
NS = 16
NW = NC * NS
ROWS_W = N // NW
CHUNK = 64
NCHUNK = ROWS_W // CHUNK
CWORDS = CHUNK * T


def _body(x_hbm, out_hbm, x_v, buf_v):
    wid = lax.axis_index("s") * NC + lax.axis_index("c")

    pltpu.sync_copy(x_hbm, x_v)

    zeros16 = jnp.zeros((16,), jnp.float32)

    def zbody(i, c):
        buf_v[pl.ds(i * 16, 16)] = zeros16
        return c

    lax.fori_loop(0, CWORDS // 16, zbody, 0, unroll=8)

    lane = lax.iota(jnp.int32, 16)
    ones16 = jnp.ones((16,), jnp.float32)
    row0 = wid * ROWS_W

    def chunk_body(ci, c):
        n0 = row0 + ci * CHUNK
        l = n0 // B
        b0 = n0 - l * B

        for g in range(CHUNK // 16):
            bg = b0 + g * 16
            xv = x_v[pl.ds(l * B + bg, 16)]
            valid = xv >= 0

            cnt = jnp.zeros((16,), jnp.int32)
            for j in range(L):
                xj = x_v[pl.ds(j * B + bg, 16)]
                hit = (xj == xv) & (j <= l)
                cnt = cnt + hit.astype(jnp.int32)
            yv = (valid & ((cnt & 1) == 0)).astype(jnp.float32)

            rowbase = (g * 16 + lane) * T
            plsc.store_scatter(buf_v, [rowbase + xv], ones16, mask=valid)
            plsc.addupdate_scatter(buf_v, [rowbase + (T - 1)], yv)

        pltpu.sync_copy(buf_v, out_hbm.at[pl.ds(n0 * T, CWORDS)])

        for g in range(CHUNK // 16):
            bg = b0 + g * 16
            xv = x_v[pl.ds(l * B + bg, 16)]
            valid = xv >= 0
            rowbase = (g * 16 + lane) * T
            plsc.store_scatter(buf_v, [rowbase + (T - 1)], zeros16)
            plsc.store_scatter(buf_v, [rowbase + xv], zeros16, mask=valid)
        return c

    lax.fori_loop(0, NCHUNK, chunk_body, 0)


_mesh = plsc.VectorSubcoreMesh(
    core_axis_name="c", subcore_axis_name="s", num_cores=NC, num_subcores=NS
)

_sc_call = pl.kernel(
    _body,
    out_type=jax.ShapeDtypeStruct((N * T,), jnp.float32),
    mesh=_mesh,
    scratch_types=[
        pltpu.VMEM((N,), jnp.int32),
        pltpu.VMEM((CWORDS,), jnp.float32),
    ],
    compiler_params=pltpu.CompilerParams(needs_layout_passes=False),
)


@jax.jit
def kernel(x):
    x32 = x.astype(jnp.int32).reshape(-1)
    out = _sc_call(x32)
    return out.reshape(L, B, T)
